# Initial kernel scaffold; baseline (speedup 1.0000x reference)
import jax
import jax.numpy as jnp
from jax.experimental import pallas as pl


def kernel(embeddings, visibility_scores, bbox_ltwh, keypoints_xyc, feats_masks, W, b):
    raise NotImplementedError("write your pallas kernel here")


# speedup vs baseline: 5.8164x; 5.8164x over previous
import jax
import jax.numpy as jnp
from jax.experimental import pallas as pl

_B, _N = 16, 2048
_D_EMB, _D_VIS, _D_BBOX, _D_KPT = 3072, 6, 4, 51
_TOKEN_DIM = 128
_ROWS = 512


def _proj_kernel(emb_ref, vis_ref, bbox_ref, kpt_ref, mask_ref,
                 wemb_ref, wvis_ref, wbbox_ref, wkpt_ref, b_ref, out_ref):
    acc = jnp.dot(emb_ref[...], wemb_ref[...],
                  preferred_element_type=jnp.float32)
    acc += jnp.dot(vis_ref[...], wvis_ref[...],
                   preferred_element_type=jnp.float32)
    acc += jnp.dot(bbox_ref[...], wbbox_ref[...],
                   preferred_element_type=jnp.float32)
    acc += jnp.dot(kpt_ref[...], wkpt_ref[...],
                   preferred_element_type=jnp.float32)
    acc += b_ref[...]
    out_ref[...] = acc * mask_ref[...]


def kernel(embeddings, visibility_scores, bbox_ltwh, keypoints_xyc,
           feats_masks, W, b):
    R = _B * _N
    emb = embeddings.reshape(R, _D_EMB)
    vis = visibility_scores.reshape(R, _D_VIS)
    bbox = bbox_ltwh.reshape(R, _D_BBOX)
    kpt = keypoints_xyc.reshape(R, _D_KPT)
    mask = feats_masks.reshape(R, 1).astype(jnp.float32)

    Wt = W.T
    wemb = Wt[:_D_EMB]
    wvis = Wt[_D_EMB:_D_EMB + _D_VIS]
    wbbox = Wt[_D_EMB + _D_VIS:_D_EMB + _D_VIS + _D_BBOX]
    wkpt = Wt[_D_EMB + _D_VIS + _D_BBOX:]
    b2 = b.reshape(1, _TOKEN_DIM)

    grid = (R // _ROWS,)
    out = pl.pallas_call(
        _proj_kernel,
        grid=grid,
        in_specs=[
            pl.BlockSpec((_ROWS, _D_EMB), lambda i: (i, 0)),
            pl.BlockSpec((_ROWS, _D_VIS), lambda i: (i, 0)),
            pl.BlockSpec((_ROWS, _D_BBOX), lambda i: (i, 0)),
            pl.BlockSpec((_ROWS, _D_KPT), lambda i: (i, 0)),
            pl.BlockSpec((_ROWS, 1), lambda i: (i, 0)),
            pl.BlockSpec((_D_EMB, _TOKEN_DIM), lambda i: (0, 0)),
            pl.BlockSpec((_D_VIS, _TOKEN_DIM), lambda i: (0, 0)),
            pl.BlockSpec((_D_BBOX, _TOKEN_DIM), lambda i: (0, 0)),
            pl.BlockSpec((_D_KPT, _TOKEN_DIM), lambda i: (0, 0)),
            pl.BlockSpec((1, _TOKEN_DIM), lambda i: (0, 0)),
        ],
        out_specs=pl.BlockSpec((_ROWS, _TOKEN_DIM), lambda i: (i, 0)),
        out_shape=jax.ShapeDtypeStruct((R, _TOKEN_DIM), jnp.float32),
    )(emb, vis, bbox, kpt, mask, wemb, wvis, wbbox, wkpt, b2)

    return out.reshape(_B, _N, _TOKEN_DIM)


# speedup vs baseline: 5.8348x; 1.0032x over previous
import jax
import jax.numpy as jnp
from jax.experimental import pallas as pl

_B, _N = 16, 2048
_D_EMB, _D_VIS, _D_BBOX, _D_KPT = 3072, 6, 4, 51
_TOKEN_DIM = 128
_ROWS = 512


def _proj_kernel(emb_ref, vis_ref, bbox_ref, kpt_ref, mask_ref,
                 wemb_ref, wvis_ref, wbbox_ref, wkpt_ref, b_ref, out_ref):
    acc = jnp.dot(emb_ref[...].astype(jnp.bfloat16), wemb_ref[...],
                  preferred_element_type=jnp.float32)
    acc += jnp.dot(vis_ref[...], wvis_ref[...],
                   preferred_element_type=jnp.float32)
    acc += jnp.dot(bbox_ref[...], wbbox_ref[...],
                   preferred_element_type=jnp.float32)
    acc += jnp.dot(kpt_ref[...], wkpt_ref[...],
                   preferred_element_type=jnp.float32)
    acc += b_ref[...]
    out_ref[...] = acc * mask_ref[...]


def kernel(embeddings, visibility_scores, bbox_ltwh, keypoints_xyc,
           feats_masks, W, b):
    R = _B * _N
    emb = embeddings.reshape(R, _D_EMB)
    vis = visibility_scores.reshape(R, _D_VIS)
    bbox = bbox_ltwh.reshape(R, _D_BBOX)
    kpt = keypoints_xyc.reshape(R, _D_KPT)
    mask = feats_masks.reshape(R, 1).astype(jnp.float32)

    Wt = W.T
    wemb = Wt[:_D_EMB].astype(jnp.bfloat16)
    wvis = Wt[_D_EMB:_D_EMB + _D_VIS]
    wbbox = Wt[_D_EMB + _D_VIS:_D_EMB + _D_VIS + _D_BBOX]
    wkpt = Wt[_D_EMB + _D_VIS + _D_BBOX:]
    b2 = b.reshape(1, _TOKEN_DIM)

    grid = (R // _ROWS,)
    out = pl.pallas_call(
        _proj_kernel,
        grid=grid,
        in_specs=[
            pl.BlockSpec((_ROWS, _D_EMB), lambda i: (i, 0)),
            pl.BlockSpec((_ROWS, _D_VIS), lambda i: (i, 0)),
            pl.BlockSpec((_ROWS, _D_BBOX), lambda i: (i, 0)),
            pl.BlockSpec((_ROWS, _D_KPT), lambda i: (i, 0)),
            pl.BlockSpec((_ROWS, 1), lambda i: (i, 0)),
            pl.BlockSpec((_D_EMB, _TOKEN_DIM), lambda i: (0, 0)),
            pl.BlockSpec((_D_VIS, _TOKEN_DIM), lambda i: (0, 0)),
            pl.BlockSpec((_D_BBOX, _TOKEN_DIM), lambda i: (0, 0)),
            pl.BlockSpec((_D_KPT, _TOKEN_DIM), lambda i: (0, 0)),
            pl.BlockSpec((1, _TOKEN_DIM), lambda i: (0, 0)),
        ],
        out_specs=pl.BlockSpec((_ROWS, _TOKEN_DIM), lambda i: (i, 0)),
        out_shape=jax.ShapeDtypeStruct((R, _TOKEN_DIM), jnp.float32),
    )(emb, vis, bbox, kpt, mask, wemb, wvis, wbbox, wkpt, b2)

    return out.reshape(_B, _N, _TOKEN_DIM)
